# Initial kernel scaffold; baseline (speedup 1.0000x reference)
import jax
import jax.numpy as jnp
from jax.experimental import pallas as pl


def kernel(logits, spec_token_ids):
    raise NotImplementedError("write your pallas kernel here")


# speedup vs baseline: 1.0350x; 1.0350x over previous
import functools

import jax
import jax.numpy as jnp
from jax.experimental import pallas as pl


def _argmax_kernel(x_ref, out_ref):
    x = x_ref[...]
    idx = jnp.argmax(x, axis=1).astype(jnp.int32)
    out_ref[...] = idx.reshape(out_ref.shape)


def _accept_kernel(ids_ref, spec_ref, sampled_ref, len_ref):
    ids = ids_ref[...]
    spec = spec_ref[...]
    B, K1 = ids.shape
    K = K1 - 1
    prod = jnp.ones((B,), dtype=jnp.int32)
    total = jnp.zeros((B,), dtype=jnp.int32)
    for k in range(K):
        m = (ids[:, k] == spec[:, k]).astype(jnp.int32)
        prod = prod * m
        total = total + prod
    lengths = total + 1
    pos = jax.lax.broadcasted_iota(jnp.int32, (B, K1), 1)
    sampled_ref[...] = jnp.where(pos < lengths[:, None], ids, -1)
    len_ref[...] = lengths.reshape(B, 1)


def kernel(logits, spec_token_ids):
    B, K = spec_token_ids.shape
    R, V = logits.shape
    ROWS = 8
    out_ids = pl.pallas_call(
        _argmax_kernel,
        grid=(R // ROWS,),
        in_specs=[pl.BlockSpec((ROWS, V), lambda i: (i, 0))],
        out_specs=pl.BlockSpec((ROWS, 1), lambda i: (i, 0)),
        out_shape=jax.ShapeDtypeStruct((R, 1), jnp.int32),
    )(logits)
    out_ids = out_ids.reshape(B, K + 1)
    sampled, lengths = pl.pallas_call(
        _accept_kernel,
        in_specs=[
            pl.BlockSpec((B, K + 1), lambda: (0, 0)),
            pl.BlockSpec((B, K), lambda: (0, 0)),
        ],
        out_specs=[
            pl.BlockSpec((B, K + 1), lambda: (0, 0)),
            pl.BlockSpec((B, 1), lambda: (0, 0)),
        ],
        out_shape=[
            jax.ShapeDtypeStruct((B, K + 1), jnp.int32),
            jax.ShapeDtypeStruct((B, 1), jnp.int32),
        ],
    )(out_ids, spec_token_ids)
    return sampled, lengths.reshape(B)


# speedup vs baseline: 1.0353x; 1.0003x over previous
import functools

import jax
import jax.numpy as jnp
from jax.experimental import pallas as pl
from jax.experimental.pallas import tpu as pltpu


def _argmax_kernel(x_ref, out_ref):
    x = x_ref[...]
    idx = jnp.argmax(x, axis=1).astype(jnp.int32)
    out_ref[...] = idx.reshape(out_ref.shape)


def _accept_kernel(ids_ref, spec_ref, sampled_ref, len_ref):
    ids = ids_ref[...]
    spec = spec_ref[...]
    B, K1 = ids.shape
    K = K1 - 1
    prod = jnp.ones((B,), dtype=jnp.int32)
    total = jnp.zeros((B,), dtype=jnp.int32)
    for k in range(K):
        m = (ids[:, k] == spec[:, k]).astype(jnp.int32)
        prod = prod * m
        total = total + prod
    lengths = total + 1
    pos = jax.lax.broadcasted_iota(jnp.int32, (B, K1), 1)
    sampled_ref[...] = jnp.where(pos < lengths[:, None], ids, -1)
    len_ref[...] = lengths.reshape(B, 1)


def kernel(logits, spec_token_ids):
    B, K = spec_token_ids.shape
    R, V = logits.shape
    ROWS = 8
    out_ids = pl.pallas_call(
        _argmax_kernel,
        grid=(R // ROWS,),
        in_specs=[pl.BlockSpec((ROWS, V), lambda i: (i, 0))],
        out_specs=pl.BlockSpec((ROWS, 1), lambda i: (i, 0)),
        out_shape=jax.ShapeDtypeStruct((R, 1), jnp.int32),
        compiler_params=pltpu.CompilerParams(
            dimension_semantics=("parallel",)),
    )(logits)
    out_ids = out_ids.reshape(B, K + 1)
    sampled, lengths = pl.pallas_call(
        _accept_kernel,
        in_specs=[
            pl.BlockSpec((B, K + 1), lambda: (0, 0)),
            pl.BlockSpec((B, K), lambda: (0, 0)),
        ],
        out_specs=[
            pl.BlockSpec((B, K + 1), lambda: (0, 0)),
            pl.BlockSpec((B, 1), lambda: (0, 0)),
        ],
        out_shape=[
            jax.ShapeDtypeStruct((B, K + 1), jnp.int32),
            jax.ShapeDtypeStruct((B, 1), jnp.int32),
        ],
    )(out_ids, spec_token_ids)
    return sampled, lengths.reshape(B)


# speedup vs baseline: 1.1742x; 1.1342x over previous
import functools

import jax
import jax.numpy as jnp
from jax.experimental import pallas as pl
from jax.experimental.pallas import tpu as pltpu


def _argmax_kernel(x_ref, out_ref):
    x = x_ref[...]
    idx = jnp.argmax(x, axis=1).astype(jnp.int32)
    out_ref[...] = idx.reshape(out_ref.shape)


def _accept_kernel(ids_ref, spec_ref, sampled_ref, len_ref):
    ids = ids_ref[...]
    spec = spec_ref[...]
    B, K1 = ids.shape
    K = K1 - 1
    prod = jnp.ones((B,), dtype=jnp.int32)
    total = jnp.zeros((B,), dtype=jnp.int32)
    for k in range(K):
        m = (ids[:, k] == spec[:, k]).astype(jnp.int32)
        prod = prod * m
        total = total + prod
    lengths = total + 1
    pos = jax.lax.broadcasted_iota(jnp.int32, (B, K1), 1)
    sampled_ref[...] = jnp.where(pos < lengths[:, None], ids, -1)
    len_ref[...] = lengths.reshape(B, 1)


def kernel(logits, spec_token_ids):
    B, K = spec_token_ids.shape
    R, V = logits.shape
    ROWS = 16
    out_ids = pl.pallas_call(
        _argmax_kernel,
        grid=(R // ROWS,),
        in_specs=[pl.BlockSpec((ROWS, V), lambda i: (i, 0))],
        out_specs=pl.BlockSpec((ROWS, 1), lambda i: (i, 0)),
        out_shape=jax.ShapeDtypeStruct((R, 1), jnp.int32),
        compiler_params=pltpu.CompilerParams(
            dimension_semantics=("parallel",)),
    )(logits)
    out_ids = out_ids.reshape(B, K + 1)
    sampled, lengths = pl.pallas_call(
        _accept_kernel,
        in_specs=[
            pl.BlockSpec((B, K + 1), lambda: (0, 0)),
            pl.BlockSpec((B, K), lambda: (0, 0)),
        ],
        out_specs=[
            pl.BlockSpec((B, K + 1), lambda: (0, 0)),
            pl.BlockSpec((B, 1), lambda: (0, 0)),
        ],
        out_shape=[
            jax.ShapeDtypeStruct((B, K + 1), jnp.int32),
            jax.ShapeDtypeStruct((B, 1), jnp.int32),
        ],
    )(out_ids, spec_token_ids)
    return sampled, lengths.reshape(B)


# speedup vs baseline: 1.3441x; 1.1447x over previous
import functools

import jax
import jax.numpy as jnp
from jax.experimental import pallas as pl
from jax.experimental.pallas import tpu as pltpu


def _argmax_kernel(x_ref, out_ref):
    x = x_ref[...]
    idx = jnp.argmax(x, axis=1).astype(jnp.int32)
    out_ref[...] = idx.reshape(out_ref.shape)


def _accept_kernel(ids_ref, spec_ref, sampled_ref, len_ref):
    ids = ids_ref[...]
    spec = spec_ref[...]
    B, K1 = ids.shape
    K = K1 - 1
    prod = jnp.ones((B,), dtype=jnp.int32)
    total = jnp.zeros((B,), dtype=jnp.int32)
    for k in range(K):
        m = (ids[:, k] == spec[:, k]).astype(jnp.int32)
        prod = prod * m
        total = total + prod
    lengths = total + 1
    pos = jax.lax.broadcasted_iota(jnp.int32, (B, K1), 1)
    sampled_ref[...] = jnp.where(pos < lengths[:, None], ids, -1)
    len_ref[...] = lengths.reshape(B, 1)


def kernel(logits, spec_token_ids):
    B, K = spec_token_ids.shape
    R, V = logits.shape
    ROWS = 32
    out_ids = pl.pallas_call(
        _argmax_kernel,
        grid=(R // ROWS,),
        in_specs=[pl.BlockSpec((ROWS, V), lambda i: (i, 0))],
        out_specs=pl.BlockSpec((ROWS, 1), lambda i: (i, 0)),
        out_shape=jax.ShapeDtypeStruct((R, 1), jnp.int32),
        compiler_params=pltpu.CompilerParams(
            dimension_semantics=("parallel",)),
    )(logits)
    out_ids = out_ids.reshape(B, K + 1)
    sampled, lengths = pl.pallas_call(
        _accept_kernel,
        in_specs=[
            pl.BlockSpec((B, K + 1), lambda: (0, 0)),
            pl.BlockSpec((B, K), lambda: (0, 0)),
        ],
        out_specs=[
            pl.BlockSpec((B, K + 1), lambda: (0, 0)),
            pl.BlockSpec((B, 1), lambda: (0, 0)),
        ],
        out_shape=[
            jax.ShapeDtypeStruct((B, K + 1), jnp.int32),
            jax.ShapeDtypeStruct((B, 1), jnp.int32),
        ],
    )(out_ids, spec_token_ids)
    return sampled, lengths.reshape(B)


# speedup vs baseline: 1.4040x; 1.0445x over previous
import functools

import jax
import jax.numpy as jnp
from jax.experimental import pallas as pl
from jax.experimental.pallas import tpu as pltpu


def _argmax_kernel(x_ref, out_ref, *, chunk):
    rows, v = x_ref.shape
    run_max = jnp.full((rows,), -jnp.inf, dtype=jnp.float32)
    run_idx = jnp.zeros((rows,), dtype=jnp.int32)
    off = 0
    while off < v:
        w = min(chunk, v - off)
        xc = x_ref[:, off:off + w]
        cmax = jnp.max(xc, axis=1)
        cidx = jnp.argmax(xc, axis=1).astype(jnp.int32) + off
        upd = cmax > run_max
        run_max = jnp.where(upd, cmax, run_max)
        run_idx = jnp.where(upd, cidx, run_idx)
        off += w
    out_ref[...] = run_idx.reshape(out_ref.shape)


def _accept_kernel(ids_ref, spec_ref, sampled_ref, len_ref):
    ids = ids_ref[...]
    spec = spec_ref[...]
    B, K1 = ids.shape
    K = K1 - 1
    prod = jnp.ones((B,), dtype=jnp.int32)
    total = jnp.zeros((B,), dtype=jnp.int32)
    for k in range(K):
        m = (ids[:, k] == spec[:, k]).astype(jnp.int32)
        prod = prod * m
        total = total + prod
    lengths = total + 1
    pos = jax.lax.broadcasted_iota(jnp.int32, (B, K1), 1)
    sampled_ref[...] = jnp.where(pos < lengths[:, None], ids, -1)
    len_ref[...] = lengths.reshape(B, 1)


def kernel(logits, spec_token_ids):
    B, K = spec_token_ids.shape
    R, V = logits.shape
    ROWS = 32
    CHUNK = 12800
    out_ids = pl.pallas_call(
        functools.partial(_argmax_kernel, chunk=CHUNK),
        grid=(R // ROWS,),
        in_specs=[pl.BlockSpec((ROWS, V), lambda i: (i, 0))],
        out_specs=pl.BlockSpec((ROWS, 1), lambda i: (i, 0)),
        out_shape=jax.ShapeDtypeStruct((R, 1), jnp.int32),
        compiler_params=pltpu.CompilerParams(
            dimension_semantics=("parallel",)),
    )(logits)
    out_ids = out_ids.reshape(B, K + 1)
    sampled, lengths = pl.pallas_call(
        _accept_kernel,
        in_specs=[
            pl.BlockSpec((B, K + 1), lambda: (0, 0)),
            pl.BlockSpec((B, K), lambda: (0, 0)),
        ],
        out_specs=[
            pl.BlockSpec((B, K + 1), lambda: (0, 0)),
            pl.BlockSpec((B, 1), lambda: (0, 0)),
        ],
        out_shape=[
            jax.ShapeDtypeStruct((B, K + 1), jnp.int32),
            jax.ShapeDtypeStruct((B, 1), jnp.int32),
        ],
    )(out_ids, spec_token_ids)
    return sampled, lengths.reshape(B)
